# baseline (speedup 1.0000x reference)
import functools

import jax
import jax.numpy as jnp
from jax import lax
from jax.experimental import pallas as pl
from jax.experimental.pallas import tpu as pltpu
from jax.experimental.pallas import tpu_sc as plsc

VOCAB = 1000000
EMBED_DIM = 64
NUM_LABELS = 50
BATCH = 4096
SEQ = 200
PAD_IDX = 3000

NC = 2
NS = 16
NW = NC * NS
BPW = BATCH // NW
SPLIT0 = 104
SPLIT1 = SEQ - SPLIT0


def _sc_pool_body(text_hbm, table_hbm, out_hbm, idx_v, rows0, rows1,
                  pooled_v, sem0, sem1):
  wid = lax.axis_index("s") * NC + lax.axis_index("c")
  base = wid * BPW

  pltpu.sync_copy(text_hbm.at[pl.ds(base, BPW)], idx_v)

  def start_row(r, buf, sem):
    pltpu.async_copy(table_hbm.at[idx_v.at[r, pl.ds(0, SPLIT0)]],
                     buf.at[pl.ds(0, SPLIT0)], sem)
    pltpu.async_copy(table_hbm.at[idx_v.at[r, pl.ds(SPLIT0, SPLIT1)]],
                     buf.at[pl.ds(SPLIT0, SPLIT1)], sem)

  def wait_buf(buf, sem):
    pltpu.make_async_copy(table_hbm.at[pl.ds(0, SEQ)], buf, sem).wait()

  def reduce_row(buf, r):
    init = tuple(jnp.full((16,), -jnp.inf, dtype=jnp.float32)
                 for _ in range(EMBED_DIM // 16))
    def blk_body(j, accs):
      accs = list(accs)
      l0 = jnp.minimum(j * 16, SEQ - 16)
      iv = idx_v[r, pl.ds(l0, 16)]
      mv = jnp.where(iv == PAD_IDX, jnp.float32(0), jnp.float32(1))
      for u in range(16):
        m = mv[u]
        for c in range(EMBED_DIM // 16):
          v = buf[l0 + u, pl.ds(c * 16, 16)]
          accs[c] = jnp.maximum(accs[c], v * m)
      return tuple(accs)
    accs = lax.fori_loop(0, (SEQ + 15) // 16, blk_body, init)
    for c in range(EMBED_DIM // 16):
      pooled_v[r, pl.ds(c * 16, 16)] = accs[c]

  start_row(0, rows0, sem0)

  def body2(i, carry):
    r = i * 2
    start_row(r + 1, rows1, sem1)
    wait_buf(rows0, sem0)
    reduce_row(rows0, r)

    @pl.when(r + 2 < BPW)
    def _():
      start_row(r + 2, rows0, sem0)

    wait_buf(rows1, sem1)
    reduce_row(rows1, r + 1)
    return carry

  lax.fori_loop(0, BPW // 2, body2, 0)
  pltpu.sync_copy(pooled_v, out_hbm.at[pl.ds(base, BPW)])


_sc_pool = functools.partial(
    pl.kernel,
    out_type=jax.ShapeDtypeStruct((BATCH, EMBED_DIM), jnp.float32),
    mesh=plsc.VectorSubcoreMesh(core_axis_name="c", subcore_axis_name="s",
                                num_cores=NC, num_subcores=NS),
    scratch_types=[
        pltpu.VMEM((BPW, SEQ), jnp.int32),
        pltpu.VMEM((SEQ, EMBED_DIM), jnp.float32),
        pltpu.VMEM((SEQ, EMBED_DIM), jnp.float32),
        pltpu.VMEM((BPW, EMBED_DIM), jnp.float32),
        pltpu.SemaphoreType.DMA,
        pltpu.SemaphoreType.DMA,
    ],
    compiler_params=pltpu.CompilerParams(use_tc_tiling_on_sc=False),
)(_sc_pool_body)


def _head_body(p_ref, wt_ref, b_ref, o_ref):
  logits = jnp.dot(p_ref[...], wt_ref[...],
                   preferred_element_type=jnp.float32) + b_ref[...]
  mx = jnp.max(logits, axis=1, keepdims=True)
  sh = logits - mx
  lse = jnp.log(jnp.sum(jnp.exp(sh), axis=1, keepdims=True))
  o_ref[...] = sh - lse


_BB = 1024

_head = pl.pallas_call(
    _head_body,
    out_shape=jax.ShapeDtypeStruct((BATCH, NUM_LABELS), jnp.float32),
    grid=(BATCH // _BB,),
    in_specs=[
        pl.BlockSpec((_BB, EMBED_DIM), lambda i: (i, 0)),
        pl.BlockSpec((EMBED_DIM, NUM_LABELS), lambda i: (0, 0)),
        pl.BlockSpec((1, NUM_LABELS), lambda i: (0, 0)),
    ],
    out_specs=pl.BlockSpec((_BB, NUM_LABELS), lambda i: (i, 0)),
)


def kernel(text, sequence_lens, table, W, b):
  del sequence_lens
  pooled = _sc_pool(text.astype(jnp.int32), table)
  return _head(pooled, W.T, b.reshape(1, NUM_LABELS))
